# Initial kernel scaffold; baseline (speedup 1.0000x reference)
import jax
import jax.numpy as jnp
from jax.experimental import pallas as pl


def kernel(h_x, t_x, h_edge_index, t_edge_index, b_edge_index, h_batch, t_batch, W0, b0, conv_Wr, conv_Wl, conv_b, mask_Ws, mask_Wn, mask_bh, mask_Wo, mask_bo, Wi, ai_s, ai_d, Wih, Wit, an_s, an_d, Wg_root, Wg_rel):
    raise NotImplementedError("write your pallas kernel here")


# speedup vs baseline: 7.8308x; 7.8308x over previous
import functools

import jax
import jax.numpy as jnp
from jax import lax
from jax.experimental import pallas as pl
from jax.experimental.pallas import tpu as pltpu
from jax.experimental.pallas import tpu_sc as plsc

N = 10000
E = 320000
D = 128
H = 128
HO = 64
G = 64
L = 3

NC = 2
NS = 16
LN = 16
NW = NC * NS
NP = 10240
RPT = NP // NS
EPW = E // NW
EB = 80
NBL = EPW // EB

_f32 = jnp.float32
_mesh = plsc.VectorSubcoreMesh(core_axis_name="c", subcore_axis_name="s")
_sc_params = pltpu.CompilerParams(needs_layout_passes=False,
                                  use_tc_tiling_on_sc=False)


def _zero_rows(buf, rows, width):
    z16 = jnp.zeros((LN,), _f32)

    @pl.loop(0, rows)
    def _(r):
        for c in range(width // LN):
            buf[r, pl.ds(c * LN, LN)] = z16


def _make_segsum(K):
    @functools.partial(
        pl.kernel,
        out_type=jax.ShapeDtypeStruct((NC, NP, K), _f32),
        mesh=_mesh,
        scratch_types=[
            pltpu.VMEM((EB,), jnp.int32),
            pltpu.VMEM((EB,), jnp.int32),
            pltpu.VMEM((EB, K), _f32),
            pltpu.VMEM_SHARED((NP, K), _f32),
            pltpu.SemaphoreType.DMA,
        ],
        compiler_params=_sc_params,
    )
    def seg(x_hbm, src_hbm, dst_hbm, out_hbm, src_v, dst_v, rows_v, acc, sem):
        cid = lax.axis_index("c")
        sid = lax.axis_index("s")
        wid = cid * NS + sid

        _zero_rows(rows_v, EB, K)
        for j in range(RPT // EB):
            pltpu.sync_copy(rows_v, acc.at[pl.ds(sid * RPT + j * EB, EB)])
        plsc.subcore_barrier()

        base0 = wid * EPW

        @pl.loop(0, NBL)
        def _(i):
            base = base0 + i * EB
            pltpu.sync_copy(src_hbm.at[pl.ds(base, EB)], src_v)
            pltpu.sync_copy(dst_hbm.at[pl.ds(base, EB)], dst_v)
            pltpu.async_copy(x_hbm.at[src_v], rows_v, sem).wait()
            pltpu.sync_copy(rows_v, acc.at[dst_v], add=True)

        plsc.subcore_barrier()
        pltpu.sync_copy(acc.at[pl.ds(sid * RPT, RPT)],
                        out_hbm.at[cid, pl.ds(sid * RPT, RPT)])

    return seg


_segsum128 = _make_segsum(H)


KA = 80


@functools.partial(
    pl.kernel,
    out_type=jax.ShapeDtypeStruct((NC, NP, KA), _f32),
    mesh=_mesh,
    scratch_types=[
        pltpu.VMEM((EB,), jnp.int32),
        pltpu.VMEM((EB,), jnp.int32),
        pltpu.VMEM((EB, HO), _f32),
        pltpu.VMEM((EB, KA), _f32),
        pltpu.VMEM((EB,), _f32),
        pltpu.VMEM((N,), _f32),
        pltpu.VMEM((N,), _f32),
        pltpu.VMEM((LN,), _f32),
        pltpu.VMEM_SHARED((NP, KA), _f32),
        pltpu.SemaphoreType.DMA,
    ],
    compiler_params=_sc_params,
)
def _sc_attn(sv_hbm, dv_hbm, gm_hbm, z_hbm, src_hbm, dst_hbm, out_hbm,
             src_v, dst_v, zrows, orows, pbuf, sv_v, dv_v, gm_v, acc, sem):
    cid = lax.axis_index("c")
    sid = lax.axis_index("s")
    wid = cid * NS + sid

    pltpu.sync_copy(sv_hbm, sv_v)
    pltpu.sync_copy(dv_hbm, dv_v)
    pltpu.sync_copy(gm_hbm, gm_v)

    _zero_rows(orows, EB, KA)
    for j in range(RPT // EB):
        pltpu.sync_copy(orows, acc.at[pl.ds(sid * RPT + j * EB, EB)])
    plsc.subcore_barrier()

    gm = gm_v[...]
    iota16 = lax.iota(jnp.int32, LN)
    base0 = wid * EPW

    @pl.loop(0, NBL)
    def _(i):
        base = base0 + i * EB
        pltpu.sync_copy(src_hbm.at[pl.ds(base, EB)], src_v)
        pltpu.sync_copy(dst_hbm.at[pl.ds(base, EB)], dst_v)
        pltpu.async_copy(z_hbm.at[src_v], zrows, sem).wait()
        for g in range(EB // LN):
            sidx = src_v[pl.ds(g * LN, LN)]
            didx = dst_v[pl.ds(g * LN, LN)]
            sv_e = plsc.load_gather(sv_v, [sidx])
            dv_e = plsc.load_gather(dv_v, [didx])
            u = sv_e + dv_e
            e = jnp.where(u >= 0.0, u, u * jnp.float32(0.2))
            mh = dv_e + gm
            mhl = jnp.where(mh >= 0.0, mh, mh * jnp.float32(0.2))
            pbuf[pl.ds(g * LN, LN)] = jnp.exp(e - mhl)

        @pl.loop(0, EB)
        def _(b):
            bidx = jnp.full((LN,), b, jnp.int32)
            pb = plsc.load_gather(pbuf, [bidx])
            for c in range(HO // LN):
                orows[b, pl.ds(c * LN, LN)] = zrows[b, pl.ds(c * LN, LN)] * pb
            orows[b, pl.ds(HO, LN)] = jnp.where(iota16 == 0, pb,
                                                jnp.float32(0.0))

        pltpu.sync_copy(orows, acc.at[dst_v], add=True)

    plsc.subcore_barrier()
    pltpu.sync_copy(acc.at[pl.ds(sid * RPT, RPT)],
                    out_hbm.at[cid, pl.ds(sid * RPT, RPT)])


BR = 2000


def _row_spec(k):
    return pl.BlockSpec((BR, k), lambda i: (i, 0))


def _full_spec(shape):
    return pl.BlockSpec(shape, lambda i: tuple(0 for _ in shape))


def _tc_in_proj(x, w, b):
    def body(x_ref, w_ref, b_ref, o_ref):
        o_ref[...] = jnp.dot(x_ref[...], w_ref[...],
                             preferred_element_type=_f32) + b_ref[...]

    return pl.pallas_call(
        body,
        grid=(N // BR,),
        in_specs=[_row_spec(D), _full_spec((D, H)), _full_spec((1, H))],
        out_specs=_row_spec(H),
        out_shape=jax.ShapeDtypeStruct((N, H), _f32),
    )(x, w, b)


def _tc_mask_pre(x, m, wn, ws):
    def body(x_ref, m_ref, wn_ref, ws_ref, yn_ref, ys_ref):
        xp = x_ref[...] * m_ref[...]
        yn_ref[...] = jnp.dot(xp, wn_ref[...], preferred_element_type=_f32)
        ys_ref[...] = jnp.dot(xp, ws_ref[...], preferred_element_type=_f32)

    return pl.pallas_call(
        body,
        grid=(N // BR,),
        in_specs=[_row_spec(H), _row_spec(1), _full_spec((H, H)),
                  _full_spec((H, H))],
        out_specs=[_row_spec(H), _row_spec(H)],
        out_shape=[jax.ShapeDtypeStruct((N, H), _f32),
                   jax.ShapeDtypeStruct((N, H), _f32)],
    )(x, m, wn, ws)


def _tc_mask_post(a0, a1, ys, x, bh, wo, bo, wl):
    def body(a0_ref, a1_ref, ys_ref, x_ref, bh_ref, wo_ref, bo_ref, wl_ref,
             m_ref, xm_ref, yl_ref):
        hmid = jax.nn.relu(ys_ref[...] + a0_ref[...] + a1_ref[...]
                           + bh_ref[...])
        m = jax.nn.sigmoid(jnp.dot(hmid, wo_ref[...],
                                   preferred_element_type=_f32) + bo_ref[...])
        xm = x_ref[...] * m
        m_ref[...] = m
        xm_ref[...] = xm
        yl_ref[...] = jnp.dot(xm, wl_ref[...], preferred_element_type=_f32)

    return pl.pallas_call(
        body,
        grid=(N // BR,),
        in_specs=[_row_spec(H), _row_spec(H), _row_spec(H), _row_spec(H),
                  _full_spec((1, H)), _full_spec((H, 1)), _full_spec((1, 1)),
                  _full_spec((H, H))],
        out_specs=[_row_spec(1), _row_spec(H), _row_spec(H)],
        out_shape=[jax.ShapeDtypeStruct((N, 1), _f32),
                   jax.ShapeDtypeStruct((N, H), _f32),
                   jax.ShapeDtypeStruct((N, H), _f32)],
    )(a0, a1, ys, x, bh, wo, bo, wl)


def _tc_conv_post(a0, a1, yl, wr, cb):
    def body(a0_ref, a1_ref, yl_ref, wr_ref, cb_ref, o_ref):
        agg = a0_ref[...] + a1_ref[...]
        o_ref[...] = jax.nn.relu(
            jnp.dot(agg, wr_ref[...], preferred_element_type=_f32)
            + yl_ref[...] + cb_ref[...])

    return pl.pallas_call(
        body,
        grid=(N // BR,),
        in_specs=[_row_spec(H), _row_spec(H), _row_spec(H),
                  _full_spec((H, H)), _full_spec((1, H))],
        out_specs=_row_spec(H),
        out_shape=jax.ShapeDtypeStruct((N, H), _f32),
    )(a0, a1, yl, wr, cb)


def _tc_mm64(x, w):
    def body(x_ref, w_ref, o_ref):
        o_ref[...] = jnp.dot(x_ref[...], w_ref[...],
                             preferred_element_type=_f32)

    return pl.pallas_call(
        body,
        grid=(N // BR,),
        in_specs=[_row_spec(H), _full_spec((H, HO))],
        out_specs=_row_spec(HO),
        out_shape=jax.ShapeDtypeStruct((N, HO), _f32),
    )(x, w)


def _tc_attn_prep(z_src, z_dst, a_s, a_d):
    def body(zs_ref, zd_ref, as_ref, ad_ref, sv_ref, dv_ref, gm_ref):
        sv = jnp.dot(zs_ref[...], as_ref[...], preferred_element_type=_f32)
        dv = jnp.dot(zd_ref[...], ad_ref[...], preferred_element_type=_f32)
        sv_ref[...] = sv
        dv_ref[...] = dv
        gm_ref[...] = jnp.max(sv).reshape(1, 1)

    return pl.pallas_call(
        body,
        grid=(1,),
        in_specs=[_full_spec((N, HO)), _full_spec((N, HO)),
                  _full_spec((HO, 1)), _full_spec((HO, 1))],
        out_specs=[_full_spec((N, 1)), _full_spec((N, 1)),
                   _full_spec((1, 1))],
        out_shape=[jax.ShapeDtypeStruct((N, 1), _f32),
                   jax.ShapeDtypeStruct((N, 1), _f32),
                   jax.ShapeDtypeStruct((1, 1), _f32)],
    )(z_src, z_dst, a_s, a_d)


def _tc_rep(gi0, gi1, ge0, ge1):
    def body(gi0_ref, gi1_ref, ge0_ref, ge1_ref, o_ref):
        eps = jnp.float32(1e-16)
        ni = gi0_ref[:, :HO] + gi1_ref[:, :HO]
        di = gi0_ref[:, HO:HO + 1] + gi1_ref[:, HO:HO + 1]
        ne = ge0_ref[:, :HO] + ge1_ref[:, :HO]
        de = ge0_ref[:, HO:HO + 1] + ge1_ref[:, HO:HO + 1]
        o_ref[...] = jnp.concatenate([ni / (di + eps), ne / (de + eps)],
                                     axis=1)

    return pl.pallas_call(
        body,
        grid=(N // BR,),
        in_specs=[_row_spec(KA)] * 4,
        out_specs=_row_spec(H),
        out_shape=jax.ShapeDtypeStruct((N, H), _f32),
    )(gi0, gi1, ge0, ge1)


def _tc_sag(rep, a0, a1, batch, wrel, wroot):
    def body(rep_ref, a0_ref, a1_ref, b_ref, wrel_ref, wroot_ref, emb_ref):
        agg = a0_ref[...] + a1_ref[...]
        sc = (jnp.dot(agg, wrel_ref[...], preferred_element_type=_f32)
              + jnp.dot(rep_ref[...], wroot_ref[...],
                        preferred_element_type=_f32))
        gid = lax.broadcasted_iota(jnp.int32, (1, G), 1)
        mask = (b_ref[...] == gid)
        maskf = mask.astype(_f32)
        neg = jnp.float32(-1e30)
        mg = jnp.max(jnp.where(mask, sc, neg), axis=0, keepdims=True)
        mn = jnp.sum(maskf * mg, axis=1, keepdims=True)
        ex = jnp.exp(sc - mn)
        sg = jnp.sum(maskf * ex, axis=0, keepdims=True)
        sn = jnp.sum(maskf * sg, axis=1, keepdims=True)
        alpha = ex / (sn + jnp.float32(1e-16))
        att = rep_ref[...] * alpha
        emb_ref[...] = lax.dot_general(maskf, att, (((0,), (0,)), ((), ())),
                                       preferred_element_type=_f32)

    return pl.pallas_call(
        body,
        grid=(1,),
        in_specs=[_full_spec((N, H)), _full_spec((N, H)), _full_spec((N, H)),
                  _full_spec((N, 1)), _full_spec((H, 1)), _full_spec((H, 1))],
        out_specs=_full_spec((G, H)),
        out_shape=jax.ShapeDtypeStruct((G, H), _f32),
    )(rep, a0, a1, batch, wrel, wroot)


_DBG_SEG = False
_DBG_ATTN = False
_DBG_TC = False


def kernel(h_x, t_x, h_edge_index, t_edge_index, b_edge_index, h_batch,
           t_batch, W0, b0, conv_Wr, conv_Wl, conv_b, mask_Ws, mask_Wn,
           mask_bh, mask_Wo, mask_bo, Wi, ai_s, ai_d, Wih, Wit, an_s, an_d,
           Wg_root, Wg_rel):
    hs = h_edge_index[0].astype(jnp.int32)
    hd = h_edge_index[1].astype(jnp.int32)
    ts = t_edge_index[0].astype(jnp.int32)
    td = t_edge_index[1].astype(jnp.int32)
    bh = b_edge_index[0].astype(jnp.int32)
    bt = b_edge_index[1].astype(jnp.int32)

    b0r = b0.reshape(1, H)
    ones = jnp.ones((N, 1), _f32)
    zcol = jnp.zeros((N, H), _f32)

    def seg(x, src, dst):
        if _DBG_SEG:
            return jax.ops.segment_sum(x[src], dst, num_segments=N), \
                jnp.zeros((N, x.shape[1]), _f32)
        out = _segsum128(x, src, dst)
        return out[0, :N], out[1, :N]

    def attn(sv_zsrc, sv_a, dv_zdst, dv_a, zrows, src, dst):
        if _DBG_ATTN:
            sv = sv_zsrc @ sv_a
            dv = dv_zdst @ dv_a
            gm = jnp.max(sv)
            u = sv[src] + dv[dst]
            e = jnp.where(u >= 0, u, 0.2 * u)
            mh = gm + dv[dst]
            mhl = jnp.where(mh >= 0, mh, 0.2 * mh)
            p = jnp.exp(e - mhl)
            num = jax.ops.segment_sum(p[:, None] * zrows[src], dst,
                                      num_segments=N)
            den = jax.ops.segment_sum(p, dst, num_segments=N)
            g0 = jnp.concatenate(
                [num, den[:, None], jnp.zeros((N, KA - HO - 1), _f32)], 1)
            return g0, jnp.zeros((N, KA), _f32)
        sv, dv, gm = _tc_attn_prep(sv_zsrc, dv_zdst, sv_a.reshape(HO, 1),
                                   dv_a.reshape(HO, 1))
        gm16 = jnp.broadcast_to(gm.reshape(1), (LN,))
        out = _sc_attn(sv.reshape(N), dv.reshape(N), gm16, zrows, src, dst)
        return out[0, :N], out[1, :N]

    def side(x_in, src, dst):
        if _DBG_TC:
            x = x_in @ W0 + b0
        else:
            x = _tc_in_proj(x_in, W0, b0r)
        m = ones
        for i in range(L):
            if _DBG_TC:
                xp = x * m
                yn = xp @ mask_Wn[i]
                ys = xp @ mask_Ws[i]
            else:
                yn, ys = _tc_mask_pre(x, m, mask_Wn[i], mask_Ws[i])
            a0, a1 = seg(yn, src, dst)
            if _DBG_TC:
                hmid = jax.nn.relu(ys + a0 + a1 + mask_bh[i])
                m = jax.nn.sigmoid(hmid @ mask_Wo[i] + mask_bo[i])
                xm = x * m
                yl = xm @ conv_Wl[i]
            else:
                m, xm, yl = _tc_mask_post(
                    a0, a1, ys, x, mask_bh[i].reshape(1, H),
                    mask_Wo[i], mask_bo[i].reshape(1, 1), conv_Wl[i])
            a20, a21 = seg(xm, src, dst)
            if _DBG_TC:
                x = jax.nn.relu((a20 + a21) @ conv_Wr[i] + yl + conv_b[i])
            else:
                x = _tc_conv_post(a20, a21, yl, conv_Wr[i],
                                  conv_b[i].reshape(1, H))
        return x

    h = side(h_x, hs, hd)
    t = side(t_x, ts, td)

    def mm64(x, w):
        return x @ w if _DBG_TC else _tc_mm64(x, w)

    def rep_of(gi0, gi1, ge0, ge1):
        if _DBG_TC:
            eps = 1e-16
            intra = (gi0[:, :HO] + gi1[:, :HO]) / (
                gi0[:, HO:HO + 1] + gi1[:, HO:HO + 1] + eps)
            inter = (ge0[:, :HO] + ge1[:, :HO]) / (
                ge0[:, HO:HO + 1] + ge1[:, HO:HO + 1] + eps)
            return jnp.concatenate([intra, inter], axis=1)
        return _tc_rep(gi0, gi1, ge0, ge1)

    def sag(rep, src, dst, batch):
        a0, a1 = seg(rep, src, dst)
        if _DBG_TC:
            agg = a0 + a1
            sc = (agg @ Wg_rel + rep @ Wg_root)[:, 0]
            m = jax.ops.segment_max(sc, batch, num_segments=G)
            m = jnp.where(jnp.isfinite(m), m, 0.0)
            ex = jnp.exp(sc - m[batch])
            s = jax.ops.segment_sum(ex, batch, num_segments=G)
            alpha = ex / (s[batch] + 1e-16)
            att = rep * alpha[:, None]
            return jax.ops.segment_sum(att, batch, num_segments=G)
        return _tc_sag(rep, a0, a1, batch.astype(jnp.int32).reshape(N, 1),
                       Wg_rel, Wg_root)

    zh_i = mm64(h, Wi)
    zt_i = mm64(t, Wi)
    gi_h0, gi_h1 = attn(zh_i, ai_s, zh_i, ai_d, zh_i, hs, hd)
    gi_t0, gi_t1 = attn(zt_i, ai_s, zt_i, ai_d, zt_i, ts, td)

    zh = mm64(h, Wih)
    zt = mm64(t, Wit)
    ge_h0, ge_h1 = attn(zt, an_s, zh, an_d, zt, bt, bh)
    ge_t0, ge_t1 = attn(zh, an_s, zt, an_d, zh, bh, bt)

    h_rep = rep_of(gi_h0, gi_h1, ge_h0, ge_h1)
    t_rep = rep_of(gi_t0, gi_t1, ge_t0, ge_t1)

    h_emb = sag(h_rep, hs, hd, h_batch)
    t_emb = sag(t_rep, ts, td, t_batch)

    return h_rep, t_rep, h_emb, t_emb
